# Initial kernel scaffold; baseline (speedup 1.0000x reference)
import jax
import jax.numpy as jnp
from jax.experimental import pallas as pl


def kernel(pred_logits, pred_boxes, target_sizes, batchsize, num_episode, num_queries, num_classes):
    raise NotImplementedError("write your pallas kernel here")


# speedup vs baseline: 2.5701x; 2.5701x over previous
import functools

import jax
import jax.numpy as jnp
from jax import lax
from jax.experimental import pallas as pl
from jax.experimental.pallas import tpu as pltpu
from jax.experimental.pallas import tpu_sc as plsc

_B, _E, _Q, _C = 32, 8, 300, 91
_ROW = _E * _Q * _C
_NBOX = _E * _Q
_K = 100
_KPAD = 112
_CH = 21840
_NCHUNK = _ROW // _CH
_SB = 1040
_NSB = _CH // _SB
_NVSB = _SB // 16
_CAPTRIG = 1040
_CAPS = 2096


def _splat(x):
  return jnp.broadcast_to(x, (16,))


_mesh = plsc.VectorSubcoreMesh(
    core_axis_name="c", subcore_axis_name="s", num_cores=2, num_subcores=16)


@functools.partial(
    pl.kernel,
    out_type=(
        jax.ShapeDtypeStruct((_B * _KPAD,), jnp.int32),
        jax.ShapeDtypeStruct((_B * _KPAD,), jnp.int32),
        jax.ShapeDtypeStruct((_B * _KPAD * 4,), jnp.float32),
    ),
    mesh=_mesh,
    compiler_params=pltpu.CompilerParams(needs_layout_passes=False),
    scratch_types=[
        pltpu.VMEM((_CH,), jnp.int32),
        pltpu.VMEM((_CAPS,), jnp.int32),
        pltpu.VMEM((_CAPS,), jnp.int32),
        pltpu.VMEM((128,), jnp.int32),
        pltpu.VMEM((128,), jnp.int32),
        pltpu.VMEM((_KPAD,), jnp.int32),
        pltpu.VMEM((_KPAD,), jnp.int32),
        pltpu.VMEM((_KPAD,), jnp.int32),
        pltpu.VMEM((_KPAD * 4,), jnp.float32),
        pltpu.VMEM((_NBOX * 4,), jnp.float32),
        pltpu.VMEM((64,), jnp.int32),
        pltpu.SemaphoreType.DMA,
    ],
)
def _postprocess_sc(prob_hbm, boxes_hbm, tsz_hbm,
                    scores_hbm, labels_hbm, oboxes_hbm,
                    sbuf, cval, cidx, tval, tidx, oval, oidx, olab, obox,
                    boxv, tsv, sem_b):
  cid = lax.axis_index("c")
  sid = lax.axis_index("s")
  b = sid * 2 + cid
  lane = lax.iota(jnp.int32, 16)

  bdesc = pltpu.async_copy(
      boxes_hbm.at[pl.ds(b * (_NBOX * 4), _NBOX * 4)], boxv, sem_b)
  pltpu.sync_copy(tsz_hbm, tsv)

  def count_pass(cur, nv, pred):
    def body(j, acc):
      k = cval[pl.ds(j * 16, 16)]
      iv = cidx[pl.ds(j * 16, 16)]
      valid = (_splat(j * 16) + lane) < _splat(cur)
      m = pred(k, iv) & valid
      return acc + jnp.sum(m.astype(jnp.int32))
    return pl.loop(0, nv, init_carry=jnp.int32(0))(body)

  def rebuild(cur, tkey):
    del tkey
    nv = (cur + 15) // 16

    def bs_body(_, lohi):
      lo, hi = lohi
      mid = lo + (hi - lo + 1) // 2
      cnt = count_pass(cur, nv, lambda k, iv: k >= _splat(mid))
      big = cnt >= _K
      return jnp.where(big, mid, lo), jnp.where(big, hi, mid - 1)
    kt, _ = pl.loop(0, 31, init_carry=(jnp.int32(0),
                                       jnp.int32(0x3F800000)))(bs_body)

    cnt_gt = count_pass(cur, nv, lambda k, iv: k >= _splat(kt + 1))
    need = _K - cnt_gt

    def bs2_body(_, lohi):
      lo, hi = lohi
      mid = (lo + hi) // 2
      cnte = count_pass(
          cur, nv,
          lambda k, iv: (k == _splat(kt)) & (iv <= _splat(mid)))
      big = cnte >= need
      return jnp.where(big, lo, mid + 1), jnp.where(big, mid, hi)
    jj, _ = pl.loop(0, 18, init_carry=(jnp.int32(0),
                                       jnp.int32(_ROW - 1)))(bs2_body)

    for j in range(7):
      tval[pl.ds(j * 16, 16)] = _splat(jnp.int32(0))
      tidx[pl.ds(j * 16, 16)] = _splat(jnp.int32(1 << 20))

    def comp_body(j, c2):
      k = cval[pl.ds(j * 16, 16)]
      iv = cidx[pl.ds(j * 16, 16)]
      valid = (_splat(j * 16) + lane) < _splat(cur)
      sel = ((k > _splat(kt)) |
             ((k == _splat(kt)) & (iv <= _splat(jj)))) & valid
      plsc.store_compressed(tval.at[pl.ds(c2, 16)], k, mask=sel)
      plsc.store_compressed(tidx.at[pl.ds(c2, 16)], iv, mask=sel)
      return c2 + jnp.sum(sel.astype(jnp.int32))
    pl.loop(0, nv, init_carry=jnp.int32(0))(comp_body)

    for j in range(7):
      cval[pl.ds(j * 16, 16)] = tval[pl.ds(j * 16, 16)]
      cidx[pl.ds(j * 16, 16)] = tidx[pl.ds(j * 16, 16)]

    return jnp.int32(_K), kt

  def no_rebuild(cur, tkey):
    return cur, tkey

  def chunk_body(k, carry):
    cur, tkey = carry
    pltpu.sync_copy(prob_hbm.at[pl.ds(b * _ROW + k * _CH, _CH)], sbuf)

    def sb_body(s, carry):
      cur, tkey = carry
      cur, tkey = lax.cond(cur > _CAPTRIG, rebuild, no_rebuild, cur, tkey)
      tvec = _splat(tkey)
      base = k * _CH + s * _SB

      def v_body(i, cur):
        v = sbuf[pl.ds(s * _SB + i * 16, 16)]
        m = v >= tvec
        plsc.store_compressed(cval.at[pl.ds(cur, 16)], v, mask=m)
        iv = _splat(base + i * 16) + lane
        plsc.store_compressed(cidx.at[pl.ds(cur, 16)], iv, mask=m)
        return cur + jnp.sum(m.astype(jnp.int32))
      cur = pl.loop(0, _NVSB, init_carry=cur)(v_body)
      return cur, tkey
    return pl.loop(0, _NSB, init_carry=(cur, tkey))(sb_body)

  init = (jnp.int32(0), jnp.int32(-(2 ** 31)))
  cur, tkey = pl.loop(0, _NCHUNK, init_carry=init)(chunk_body)

  rebuild(cur, tkey)

  for j in range(7):
    oval[pl.ds(j * 16, 16)] = _splat(jnp.int32(0))
    oidx[pl.ds(j * 16, 16)] = _splat(jnp.int32(0))

  for r in range(7):
    ki = tval[pl.ds(r * 16, 16)]
    ii = tidx[pl.ds(r * 16, 16)]

    def rk_body(j, acc):
      bk = plsc.load_gather(tval, [_splat(j)])
      bi = plsc.load_gather(tidx, [_splat(j)])
      beats = (bk > ki) | ((bk == ki) & (bi < ii))
      return acc + beats.astype(jnp.int32)
    rank = pl.loop(0, _K, init_carry=_splat(jnp.int32(0)))(rk_body)
    mask = (rank < _K) & ((_splat(r * 16) + lane) < _splat(jnp.int32(_K)))
    plsc.store_scatter(oval, [rank], ki, mask=mask)
    plsc.store_scatter(oidx, [rank], ii, mask=mask)

  bdesc.wait()
  sw = plsc.load_gather(tsv, [_splat(2 * b + 1)]).astype(jnp.float32)
  sh = plsc.load_gather(tsv, [_splat(2 * b)]).astype(jnp.float32)
  recip_c = jnp.float32(1.0 / _C)
  for r in range(7):
    iv = oidx[pl.ds(r * 16, 16)]
    xf = iv.astype(jnp.float32)
    q = (xf * recip_c + jnp.float32(0.005)).astype(jnp.int32)
    olab[pl.ds(r * 16, 16)] = iv - q * _C
    bq = q * 4
    cx = plsc.load_gather(boxv, [bq])
    cy = plsc.load_gather(boxv, [bq + 1])
    w = plsc.load_gather(boxv, [bq + 2])
    h = plsc.load_gather(boxv, [bq + 3])
    x0 = (cx - 0.5 * w) * sw
    y0 = (cy - 0.5 * h) * sh
    x1 = (cx + 0.5 * w) * sw
    y1 = (cy + 0.5 * h) * sh
    obox[pl.ds(0 * _KPAD + r * 16, 16)] = x0
    obox[pl.ds(1 * _KPAD + r * 16, 16)] = y0
    obox[pl.ds(2 * _KPAD + r * 16, 16)] = x1
    obox[pl.ds(3 * _KPAD + r * 16, 16)] = y1

  pltpu.sync_copy(oval, scores_hbm.at[pl.ds(b * _KPAD, _KPAD)])
  pltpu.sync_copy(olab, labels_hbm.at[pl.ds(b * _KPAD, _KPAD)])
  pltpu.sync_copy(obox, oboxes_hbm.at[pl.ds(b * _KPAD * 4, _KPAD * 4)])


def kernel(pred_logits, pred_boxes, target_sizes, batchsize, num_episode,
           num_queries, num_classes):
  del batchsize, num_episode, num_queries, num_classes
  prob = jax.nn.sigmoid(pred_logits).reshape(-1)
  keys = lax.bitcast_convert_type(prob, jnp.int32)
  boxes_flat = pred_boxes.reshape(-1)
  tsz = target_sizes.reshape(-1).astype(jnp.int32)
  scores_f, labels_f, boxes_f = _postprocess_sc(keys, boxes_flat, tsz)
  scores = lax.bitcast_convert_type(scores_f, jnp.float32).reshape(
      _B, _KPAD)[:, :_K]
  labels = labels_f.reshape(_B, _KPAD)[:, :_K]
  boxes = boxes_f.reshape(_B, 4, _KPAD).transpose(0, 2, 1)[:, :_K, :]
  return scores, labels, boxes


# speedup vs baseline: 2.6137x; 1.0170x over previous
import functools

import jax
import jax.numpy as jnp
from jax import lax
from jax.experimental import pallas as pl
from jax.experimental.pallas import tpu as pltpu
from jax.experimental.pallas import tpu_sc as plsc

_B, _E, _Q, _C = 32, 8, 300, 91
_ROW = _E * _Q * _C
_NBOX = _E * _Q
_K = 100
_KPAD = 112
_CH = 21840
_NCHUNK = _ROW // _CH
_SB = 1040
_NSB = _CH // _SB
_NVSB = _SB // 16
_CAPTRIG = 1040
_CAPS = 2176


def _splat(x):
  return jnp.broadcast_to(x, (16,))


_mesh = plsc.VectorSubcoreMesh(
    core_axis_name="c", subcore_axis_name="s", num_cores=2, num_subcores=16)


@functools.partial(
    pl.kernel,
    out_type=(
        jax.ShapeDtypeStruct((_B * _KPAD,), jnp.int32),
        jax.ShapeDtypeStruct((_B * _KPAD,), jnp.int32),
        jax.ShapeDtypeStruct((_B * _KPAD * 4,), jnp.float32),
    ),
    mesh=_mesh,
    compiler_params=pltpu.CompilerParams(needs_layout_passes=False),
    scratch_types=[
        pltpu.VMEM((_CH,), jnp.int32),
        pltpu.VMEM((_CAPS,), jnp.int32),
        pltpu.VMEM((_CAPS,), jnp.int32),
        pltpu.VMEM((128,), jnp.int32),
        pltpu.VMEM((128,), jnp.int32),
        pltpu.VMEM((_KPAD,), jnp.int32),
        pltpu.VMEM((_KPAD,), jnp.int32),
        pltpu.VMEM((_KPAD,), jnp.int32),
        pltpu.VMEM((_KPAD * 4,), jnp.float32),
        pltpu.VMEM((_NBOX * 4,), jnp.float32),
        pltpu.VMEM((64,), jnp.int32),
        pltpu.SemaphoreType.DMA,
    ],
)
def _postprocess_sc(prob_hbm, boxes_hbm, tsz_hbm,
                    scores_hbm, labels_hbm, oboxes_hbm,
                    sbuf, cval, cidx, tval, tidx, oval, oidx, olab, obox,
                    boxv, tsv, sem_b):
  cid = lax.axis_index("c")
  sid = lax.axis_index("s")
  b = sid * 2 + cid
  lane = lax.iota(jnp.int32, 16)

  bdesc = pltpu.async_copy(
      boxes_hbm.at[pl.ds(b * (_NBOX * 4), _NBOX * 4)], boxv, sem_b)
  pltpu.sync_copy(tsz_hbm, tsv)

  def count_pass(cur, nv4, pred):
    def body(j4, acc):
      for u in range(4):
        j = j4 * 4 + u
        k = cval[pl.ds(j * 16, 16)]
        iv = cidx[pl.ds(j * 16, 16)]
        valid = (_splat(j * 16) + lane) < _splat(cur)
        m = pred(k, iv) & valid
        acc = acc + jnp.sum(m.astype(jnp.int32))
      return acc
    return pl.loop(0, nv4, init_carry=jnp.int32(0))(body)

  def rebuild(cur, tkey):
    del tkey
    nv = (cur + 15) // 16
    nv4 = (cur + 63) // 64

    def bs_body(_, lohi):
      lo, hi = lohi
      mid = lo + (hi - lo + 1) // 2
      cnt = count_pass(cur, nv4, lambda k, iv: k >= _splat(mid))
      big = cnt >= _K
      return jnp.where(big, mid, lo), jnp.where(big, hi, mid - 1)
    kt, _ = pl.loop(0, 31, init_carry=(jnp.int32(0),
                                       jnp.int32(0x3F800000)))(bs_body)

    cnt_gt = count_pass(cur, nv4, lambda k, iv: k >= _splat(kt + 1))
    need = _K - cnt_gt

    def bs2_body(_, lohi):
      lo, hi = lohi
      mid = (lo + hi) // 2
      cnte = count_pass(
          cur, nv4,
          lambda k, iv: (k == _splat(kt)) & (iv <= _splat(mid)))
      big = cnte >= need
      return jnp.where(big, lo, mid + 1), jnp.where(big, mid, hi)
    jj, _ = pl.loop(0, 18, init_carry=(jnp.int32(0),
                                       jnp.int32(_ROW - 1)))(bs2_body)

    for j in range(7):
      tval[pl.ds(j * 16, 16)] = _splat(jnp.int32(0))
      tidx[pl.ds(j * 16, 16)] = _splat(jnp.int32(1 << 20))

    def comp_body(j, c2):
      k = cval[pl.ds(j * 16, 16)]
      iv = cidx[pl.ds(j * 16, 16)]
      valid = (_splat(j * 16) + lane) < _splat(cur)
      sel = ((k > _splat(kt)) |
             ((k == _splat(kt)) & (iv <= _splat(jj)))) & valid
      plsc.store_compressed(tval.at[pl.ds(c2, 16)], k, mask=sel)
      plsc.store_compressed(tidx.at[pl.ds(c2, 16)], iv, mask=sel)
      return c2 + jnp.sum(sel.astype(jnp.int32))
    pl.loop(0, nv, init_carry=jnp.int32(0))(comp_body)

    for j in range(7):
      cval[pl.ds(j * 16, 16)] = tval[pl.ds(j * 16, 16)]
      cidx[pl.ds(j * 16, 16)] = tidx[pl.ds(j * 16, 16)]

    return jnp.int32(_K), kt

  def no_rebuild(cur, tkey):
    return cur, tkey

  def chunk_body(k, carry):
    cur, tkey = carry
    pltpu.sync_copy(prob_hbm.at[pl.ds(b * _ROW + k * _CH, _CH)], sbuf)

    def sb_body(s, carry):
      cur, tkey = carry
      cur, tkey = lax.cond(cur > _CAPTRIG, rebuild, no_rebuild, cur, tkey)
      tvec = _splat(tkey)
      base = k * _CH + s * _SB

      def v_body(i, cur):
        v = sbuf[pl.ds(s * _SB + i * 16, 16)]
        m = v >= tvec
        plsc.store_compressed(cval.at[pl.ds(cur, 16)], v, mask=m)
        iv = _splat(base + i * 16) + lane
        plsc.store_compressed(cidx.at[pl.ds(cur, 16)], iv, mask=m)
        return cur + jnp.sum(m.astype(jnp.int32))
      cur = pl.loop(0, _NVSB, init_carry=cur, unroll=8)(v_body)
      return cur, tkey
    return pl.loop(0, _NSB, init_carry=(cur, tkey))(sb_body)

  init = (jnp.int32(0), jnp.int32(-(2 ** 31)))
  cur, tkey = pl.loop(0, _NCHUNK, init_carry=init)(chunk_body)

  rebuild(cur, tkey)

  for j in range(7):
    oval[pl.ds(j * 16, 16)] = _splat(jnp.int32(0))
    oidx[pl.ds(j * 16, 16)] = _splat(jnp.int32(0))

  for r in range(7):
    ki = tval[pl.ds(r * 16, 16)]
    ii = tidx[pl.ds(r * 16, 16)]

    def rk_body(j, acc):
      bk = plsc.load_gather(tval, [_splat(j)])
      bi = plsc.load_gather(tidx, [_splat(j)])
      beats = (bk > ki) | ((bk == ki) & (bi < ii))
      return acc + beats.astype(jnp.int32)
    rank = pl.loop(0, _K, init_carry=_splat(jnp.int32(0)), unroll=4)(rk_body)
    mask = (rank < _K) & ((_splat(r * 16) + lane) < _splat(jnp.int32(_K)))
    plsc.store_scatter(oval, [rank], ki, mask=mask)
    plsc.store_scatter(oidx, [rank], ii, mask=mask)

  bdesc.wait()
  sw = plsc.load_gather(tsv, [_splat(2 * b + 1)]).astype(jnp.float32)
  sh = plsc.load_gather(tsv, [_splat(2 * b)]).astype(jnp.float32)
  recip_c = jnp.float32(1.0 / _C)
  for r in range(7):
    iv = oidx[pl.ds(r * 16, 16)]
    xf = iv.astype(jnp.float32)
    q = (xf * recip_c + jnp.float32(0.005)).astype(jnp.int32)
    olab[pl.ds(r * 16, 16)] = iv - q * _C
    bq = q * 4
    cx = plsc.load_gather(boxv, [bq])
    cy = plsc.load_gather(boxv, [bq + 1])
    w = plsc.load_gather(boxv, [bq + 2])
    h = plsc.load_gather(boxv, [bq + 3])
    x0 = (cx - 0.5 * w) * sw
    y0 = (cy - 0.5 * h) * sh
    x1 = (cx + 0.5 * w) * sw
    y1 = (cy + 0.5 * h) * sh
    obox[pl.ds(0 * _KPAD + r * 16, 16)] = x0
    obox[pl.ds(1 * _KPAD + r * 16, 16)] = y0
    obox[pl.ds(2 * _KPAD + r * 16, 16)] = x1
    obox[pl.ds(3 * _KPAD + r * 16, 16)] = y1

  pltpu.sync_copy(oval, scores_hbm.at[pl.ds(b * _KPAD, _KPAD)])
  pltpu.sync_copy(olab, labels_hbm.at[pl.ds(b * _KPAD, _KPAD)])
  pltpu.sync_copy(obox, oboxes_hbm.at[pl.ds(b * _KPAD * 4, _KPAD * 4)])


def kernel(pred_logits, pred_boxes, target_sizes, batchsize, num_episode,
           num_queries, num_classes):
  del batchsize, num_episode, num_queries, num_classes
  prob = jax.nn.sigmoid(pred_logits).reshape(-1)
  keys = lax.bitcast_convert_type(prob, jnp.int32)
  boxes_flat = pred_boxes.reshape(-1)
  tsz = target_sizes.reshape(-1).astype(jnp.int32)
  scores_f, labels_f, boxes_f = _postprocess_sc(keys, boxes_flat, tsz)
  scores = lax.bitcast_convert_type(scores_f, jnp.float32).reshape(
      _B, _KPAD)[:, :_K]
  labels = labels_f.reshape(_B, _KPAD)[:, :_K]
  boxes = boxes_f.reshape(_B, 4, _KPAD).transpose(0, 2, 1)[:, :_K, :]
  return scores, labels, boxes


# speedup vs baseline: 2.7882x; 1.0668x over previous
import functools

import jax
import jax.numpy as jnp
from jax import lax
from jax.experimental import pallas as pl
from jax.experimental.pallas import tpu as pltpu
from jax.experimental.pallas import tpu_sc as plsc

_B, _E, _Q, _C = 32, 8, 300, 91
_ROW = _E * _Q * _C
_NBOX = _E * _Q
_K = 100
_KPAD = 112
_CH = 21840
_NCHUNK = _ROW // _CH
_SB = 1040
_NSB = _CH // _SB
_NVSB = _SB // 16
_CAPTRIG = 1040
_CAPS = 2176


def _splat(x):
  return jnp.broadcast_to(x, (16,))


_mesh = plsc.VectorSubcoreMesh(
    core_axis_name="c", subcore_axis_name="s", num_cores=2, num_subcores=16)


@functools.partial(
    pl.kernel,
    out_type=(
        jax.ShapeDtypeStruct((_B * _KPAD,), jnp.int32),
        jax.ShapeDtypeStruct((_B * _KPAD,), jnp.int32),
        jax.ShapeDtypeStruct((_B * _KPAD * 4,), jnp.float32),
    ),
    mesh=_mesh,
    compiler_params=pltpu.CompilerParams(needs_layout_passes=False),
    scratch_types=[
        pltpu.VMEM((2 * _CH,), jnp.int32),
        pltpu.VMEM((_CAPS,), jnp.int32),
        pltpu.VMEM((_CAPS,), jnp.int32),
        pltpu.VMEM((128,), jnp.int32),
        pltpu.VMEM((128,), jnp.int32),
        pltpu.VMEM((_KPAD,), jnp.int32),
        pltpu.VMEM((_KPAD,), jnp.int32),
        pltpu.VMEM((_KPAD,), jnp.int32),
        pltpu.VMEM((_KPAD * 4,), jnp.float32),
        pltpu.VMEM((_NBOX * 4,), jnp.float32),
        pltpu.VMEM((64,), jnp.int32),
        pltpu.SemaphoreType.DMA,
        pltpu.SemaphoreType.DMA,
    ],
)
def _postprocess_sc(prob_hbm, boxes_hbm, tsz_hbm,
                    scores_hbm, labels_hbm, oboxes_hbm,
                    sbuf, cval, cidx, tval, tidx, oval, oidx, olab, obox,
                    boxv, tsv, sem_b, sem_s):
  cid = lax.axis_index("c")
  sid = lax.axis_index("s")
  b = sid * 2 + cid
  lane = lax.iota(jnp.int32, 16)

  bdesc = pltpu.async_copy(
      boxes_hbm.at[pl.ds(b * (_NBOX * 4), _NBOX * 4)], boxv, sem_b)
  pltpu.sync_copy(tsz_hbm, tsv)

  def count_pass(cur, nv4, pred):
    def body(j4, acc):
      for u in range(4):
        j = j4 * 4 + u
        k = cval[pl.ds(j * 16, 16)]
        iv = cidx[pl.ds(j * 16, 16)]
        valid = (_splat(j * 16) + lane) < _splat(cur)
        m = pred(k, iv) & valid
        acc = acc + plsc.all_reduce_population_count(m)[0]
      return acc
    return pl.loop(0, nv4, init_carry=jnp.int32(0))(body)

  def rebuild(cur, tkey):
    del tkey
    nv = (cur + 15) // 16
    nv4 = (cur + 63) // 64

    def bs_body(_, lohi):
      lo, hi = lohi
      mid = lo + (hi - lo + 1) // 2
      cnt = count_pass(cur, nv4, lambda k, iv: k >= _splat(mid))
      big = cnt >= _K
      return jnp.where(big, mid, lo), jnp.where(big, hi, mid - 1)
    kt, _ = pl.loop(0, 31, init_carry=(jnp.int32(0),
                                       jnp.int32(0x3F800000)))(bs_body)

    cnt_gt = count_pass(cur, nv4, lambda k, iv: k >= _splat(kt + 1))
    need = _K - cnt_gt

    def bs2_body(_, lohi):
      lo, hi = lohi
      mid = (lo + hi) // 2
      cnte = count_pass(
          cur, nv4,
          lambda k, iv: (k == _splat(kt)) & (iv <= _splat(mid)))
      big = cnte >= need
      return jnp.where(big, lo, mid + 1), jnp.where(big, mid, hi)
    jj, _ = pl.loop(0, 18, init_carry=(jnp.int32(0),
                                       jnp.int32(_ROW - 1)))(bs2_body)

    for j in range(7):
      tval[pl.ds(j * 16, 16)] = _splat(jnp.int32(0))
      tidx[pl.ds(j * 16, 16)] = _splat(jnp.int32(1 << 20))

    def comp_body(j, c2):
      k = cval[pl.ds(j * 16, 16)]
      iv = cidx[pl.ds(j * 16, 16)]
      valid = (_splat(j * 16) + lane) < _splat(cur)
      sel = ((k > _splat(kt)) |
             ((k == _splat(kt)) & (iv <= _splat(jj)))) & valid
      plsc.store_compressed(tval.at[pl.ds(c2, 16)], k, mask=sel)
      plsc.store_compressed(tidx.at[pl.ds(c2, 16)], iv, mask=sel)
      return c2 + plsc.all_reduce_population_count(sel)[0]
    pl.loop(0, nv, init_carry=jnp.int32(0))(comp_body)

    for j in range(7):
      cval[pl.ds(j * 16, 16)] = tval[pl.ds(j * 16, 16)]
      cidx[pl.ds(j * 16, 16)] = tidx[pl.ds(j * 16, 16)]

    return jnp.int32(_K), kt

  def no_rebuild(cur, tkey):
    return cur, tkey

  pltpu.async_copy(
      prob_hbm.at[pl.ds(b * _ROW, _CH)], sbuf.at[pl.ds(0, _CH)], sem_s)

  def chunk_body(k, carry):
    cur, tkey = carry
    pltpu.make_async_copy(
        prob_hbm.at[pl.ds(0, _CH)], sbuf.at[pl.ds(0, _CH)], sem_s).wait()

    @pl.when(k + 1 < _NCHUNK)
    def _prefetch():
      pltpu.async_copy(
          prob_hbm.at[pl.ds(b * _ROW + (k + 1) * _CH, _CH)],
          sbuf.at[pl.ds(((k + 1) % 2) * _CH, _CH)], sem_s)

    slot_off = (k % 2) * _CH

    def sb_body(s, carry):
      cur, tkey = carry
      cur, tkey = lax.cond(cur > _CAPTRIG, rebuild, no_rebuild, cur, tkey)
      tvec = _splat(tkey)
      base = k * _CH + s * _SB

      def v_body(i, cur):
        v = sbuf[pl.ds(slot_off + s * _SB + i * 16, 16)]
        m = v >= tvec
        plsc.store_compressed(cval.at[pl.ds(cur, 16)], v, mask=m)
        iv = _splat(base + i * 16) + lane
        plsc.store_compressed(cidx.at[pl.ds(cur, 16)], iv, mask=m)
        return cur + plsc.all_reduce_population_count(m)[0]
      cur = pl.loop(0, _NVSB, init_carry=cur, unroll=8)(v_body)
      return cur, tkey
    return pl.loop(0, _NSB, init_carry=(cur, tkey))(sb_body)

  init = (jnp.int32(0), jnp.int32(-(2 ** 31)))
  cur, tkey = pl.loop(0, _NCHUNK, init_carry=init)(chunk_body)

  rebuild(cur, tkey)

  for j in range(7):
    oval[pl.ds(j * 16, 16)] = _splat(jnp.int32(0))
    oidx[pl.ds(j * 16, 16)] = _splat(jnp.int32(0))

  for r in range(7):
    ki = tval[pl.ds(r * 16, 16)]
    ii = tidx[pl.ds(r * 16, 16)]

    def rk_body(j, acc):
      bk = plsc.load_gather(tval, [_splat(j)])
      bi = plsc.load_gather(tidx, [_splat(j)])
      beats = (bk > ki) | ((bk == ki) & (bi < ii))
      return acc + beats.astype(jnp.int32)
    rank = pl.loop(0, _K, init_carry=_splat(jnp.int32(0)), unroll=4)(rk_body)
    mask = (rank < _K) & ((_splat(r * 16) + lane) < _splat(jnp.int32(_K)))
    plsc.store_scatter(oval, [rank], ki, mask=mask)
    plsc.store_scatter(oidx, [rank], ii, mask=mask)

  bdesc.wait()
  sw = plsc.load_gather(tsv, [_splat(2 * b + 1)]).astype(jnp.float32)
  sh = plsc.load_gather(tsv, [_splat(2 * b)]).astype(jnp.float32)
  recip_c = jnp.float32(1.0 / _C)
  for r in range(7):
    iv = oidx[pl.ds(r * 16, 16)]
    xf = iv.astype(jnp.float32)
    q = (xf * recip_c + jnp.float32(0.005)).astype(jnp.int32)
    olab[pl.ds(r * 16, 16)] = iv - q * _C
    bq = q * 4
    cx = plsc.load_gather(boxv, [bq])
    cy = plsc.load_gather(boxv, [bq + 1])
    w = plsc.load_gather(boxv, [bq + 2])
    h = plsc.load_gather(boxv, [bq + 3])
    x0 = (cx - 0.5 * w) * sw
    y0 = (cy - 0.5 * h) * sh
    x1 = (cx + 0.5 * w) * sw
    y1 = (cy + 0.5 * h) * sh
    obox[pl.ds(0 * _KPAD + r * 16, 16)] = x0
    obox[pl.ds(1 * _KPAD + r * 16, 16)] = y0
    obox[pl.ds(2 * _KPAD + r * 16, 16)] = x1
    obox[pl.ds(3 * _KPAD + r * 16, 16)] = y1

  pltpu.sync_copy(oval, scores_hbm.at[pl.ds(b * _KPAD, _KPAD)])
  pltpu.sync_copy(olab, labels_hbm.at[pl.ds(b * _KPAD, _KPAD)])
  pltpu.sync_copy(obox, oboxes_hbm.at[pl.ds(b * _KPAD * 4, _KPAD * 4)])


def kernel(pred_logits, pred_boxes, target_sizes, batchsize, num_episode,
           num_queries, num_classes):
  del batchsize, num_episode, num_queries, num_classes
  prob = jax.nn.sigmoid(pred_logits).reshape(-1)
  keys = lax.bitcast_convert_type(prob, jnp.int32)
  boxes_flat = pred_boxes.reshape(-1)
  tsz = target_sizes.reshape(-1).astype(jnp.int32)
  scores_f, labels_f, boxes_f = _postprocess_sc(keys, boxes_flat, tsz)
  scores = lax.bitcast_convert_type(scores_f, jnp.float32).reshape(
      _B, _KPAD)[:, :_K]
  labels = labels_f.reshape(_B, _KPAD)[:, :_K]
  boxes = boxes_f.reshape(_B, 4, _KPAD).transpose(0, 2, 1)[:, :_K, :]
  return scores, labels, boxes


# speedup vs baseline: 2.8780x; 1.0322x over previous
import functools

import jax
import jax.numpy as jnp
from jax import lax
from jax.experimental import pallas as pl
from jax.experimental.pallas import tpu as pltpu
from jax.experimental.pallas import tpu_sc as plsc

_B, _E, _Q, _C = 32, 8, 300, 91
_ROW = _E * _Q * _C
_NBOX = _E * _Q
_K = 100
_KPAD = 112
_CH = 21840
_NCHUNK = _ROW // _CH
_SB = 1040
_NSB = _CH // _SB
_NVSB = _SB // 16
_CAPTRIG = 1040
_CAPS = 2176


def _splat(x):
  return jnp.broadcast_to(x, (16,))


_mesh = plsc.VectorSubcoreMesh(
    core_axis_name="c", subcore_axis_name="s", num_cores=2, num_subcores=16)


@functools.partial(
    pl.kernel,
    out_type=(
        jax.ShapeDtypeStruct((_B * _KPAD,), jnp.int32),
        jax.ShapeDtypeStruct((_B * _KPAD,), jnp.int32),
        jax.ShapeDtypeStruct((_B * _KPAD * 4,), jnp.float32),
    ),
    mesh=_mesh,
    compiler_params=pltpu.CompilerParams(needs_layout_passes=False),
    scratch_types=[
        pltpu.VMEM((2 * _CH,), jnp.int32),
        pltpu.VMEM((_CAPS,), jnp.int32),
        pltpu.VMEM((_CAPS,), jnp.int32),
        pltpu.VMEM((128,), jnp.int32),
        pltpu.VMEM((128,), jnp.int32),
        pltpu.VMEM((_KPAD,), jnp.int32),
        pltpu.VMEM((_KPAD,), jnp.int32),
        pltpu.VMEM((_KPAD,), jnp.int32),
        pltpu.VMEM((_KPAD * 4,), jnp.float32),
        pltpu.VMEM((_NBOX * 4,), jnp.float32),
        pltpu.VMEM((64,), jnp.int32),
        pltpu.SemaphoreType.DMA,
        pltpu.SemaphoreType.DMA,
    ],
)
def _postprocess_sc(prob_hbm, boxes_hbm, tsz_hbm,
                    scores_hbm, labels_hbm, oboxes_hbm,
                    sbuf, cval, cidx, tval, tidx, oval, oidx, olab, obox,
                    boxv, tsv, sem_b, sem_s):
  cid = lax.axis_index("c")
  sid = lax.axis_index("s")
  b = sid * 2 + cid
  lane = lax.iota(jnp.int32, 16)

  bdesc = pltpu.async_copy(
      boxes_hbm.at[pl.ds(b * (_NBOX * 4), _NBOX * 4)], boxv, sem_b)
  pltpu.sync_copy(tsz_hbm, tsv)

  def count_pass(cur, nv4, pred):
    def body(j4, acc):
      for u in range(4):
        j = j4 * 4 + u
        k = cval[pl.ds(j * 16, 16)]
        iv = cidx[pl.ds(j * 16, 16)]
        valid = (_splat(j * 16) + lane) < _splat(cur)
        m = pred(k, iv) & valid
        acc = acc + plsc.all_reduce_population_count(m)[0]
      return acc
    return pl.loop(0, nv4, init_carry=jnp.int32(0))(body)

  def rebuild(cur, tkey):
    del tkey
    nv = (cur + 15) // 16
    nv4 = (cur + 63) // 64

    def bs_body(_, lohi):
      lo, hi = lohi
      mid = lo + (hi - lo + 1) // 2
      cnt = count_pass(cur, nv4, lambda k, iv: k >= _splat(mid))
      big = cnt >= _K
      return jnp.where(big, mid, lo), jnp.where(big, hi, mid - 1)
    kt, _ = pl.loop(0, 31, init_carry=(jnp.int32(0),
                                       jnp.int32(0x3F800000)))(bs_body)

    cnt_gt = count_pass(cur, nv4, lambda k, iv: k >= _splat(kt + 1))
    need = _K - cnt_gt

    def bs2_body(_, lohi):
      lo, hi = lohi
      mid = (lo + hi) // 2
      cnte = count_pass(
          cur, nv4,
          lambda k, iv: (k == _splat(kt)) & (iv <= _splat(mid)))
      big = cnte >= need
      return jnp.where(big, lo, mid + 1), jnp.where(big, mid, hi)
    jj, _ = pl.loop(0, 18, init_carry=(jnp.int32(0),
                                       jnp.int32(_ROW - 1)))(bs2_body)

    for j in range(7):
      tval[pl.ds(j * 16, 16)] = _splat(jnp.int32(0))
      tidx[pl.ds(j * 16, 16)] = _splat(jnp.int32(1 << 20))

    def comp_body(j, c2):
      k = cval[pl.ds(j * 16, 16)]
      iv = cidx[pl.ds(j * 16, 16)]
      valid = (_splat(j * 16) + lane) < _splat(cur)
      sel = ((k > _splat(kt)) |
             ((k == _splat(kt)) & (iv <= _splat(jj)))) & valid
      plsc.store_compressed(tval.at[pl.ds(c2, 16)], k, mask=sel)
      plsc.store_compressed(tidx.at[pl.ds(c2, 16)], iv, mask=sel)
      return c2 + plsc.all_reduce_population_count(sel)[0]
    pl.loop(0, nv, init_carry=jnp.int32(0))(comp_body)

    for j in range(7):
      cval[pl.ds(j * 16, 16)] = tval[pl.ds(j * 16, 16)]
      cidx[pl.ds(j * 16, 16)] = tidx[pl.ds(j * 16, 16)]

    return jnp.int32(_K), kt

  def no_rebuild(cur, tkey):
    return cur, tkey

  pltpu.async_copy(
      prob_hbm.at[pl.ds(b * _ROW, _CH)], sbuf.at[pl.ds(0, _CH)], sem_s)

  def chunk_body(k, carry):
    cur, tkey = carry
    pltpu.make_async_copy(
        prob_hbm.at[pl.ds(0, _CH)], sbuf.at[pl.ds(0, _CH)], sem_s).wait()

    @pl.when(k + 1 < _NCHUNK)
    def _prefetch():
      pltpu.async_copy(
          prob_hbm.at[pl.ds(b * _ROW + (k + 1) * _CH, _CH)],
          sbuf.at[pl.ds(((k + 1) % 2) * _CH, _CH)], sem_s)

    slot_off = (k % 2) * _CH

    def sb_body(s, carry):
      cur, tkey = carry
      cur, tkey = lax.cond(cur > _CAPTRIG, rebuild, no_rebuild, cur, tkey)
      tvec = _splat(tkey)
      base = k * _CH + s * _SB
      sb0 = slot_off + s * _SB
      offs = []
      off = cur
      for i in range(_NVSB):
        m = sbuf[pl.ds(sb0 + i * 16, 16)] >= tvec
        offs.append(off)
        off = off + plsc.all_reduce_population_count(m)[0]
      for i in range(_NVSB):
        v = sbuf[pl.ds(sb0 + i * 16, 16)]
        m = v >= tvec
        plsc.store_compressed(cval.at[pl.ds(offs[i], 16)], v, mask=m)
        iv = _splat(base + i * 16) + lane
        plsc.store_compressed(cidx.at[pl.ds(offs[i], 16)], iv, mask=m)
      return off, tkey
    return pl.loop(0, _NSB, init_carry=(cur, tkey))(sb_body)

  init = (jnp.int32(0), jnp.int32(-(2 ** 31)))
  cur, tkey = pl.loop(0, _NCHUNK, init_carry=init)(chunk_body)

  rebuild(cur, tkey)

  for j in range(7):
    oval[pl.ds(j * 16, 16)] = _splat(jnp.int32(0))
    oidx[pl.ds(j * 16, 16)] = _splat(jnp.int32(0))

  for r in range(7):
    ki = tval[pl.ds(r * 16, 16)]
    ii = tidx[pl.ds(r * 16, 16)]

    def rk_body(j, acc):
      bk = plsc.load_gather(tval, [_splat(j)])
      bi = plsc.load_gather(tidx, [_splat(j)])
      beats = (bk > ki) | ((bk == ki) & (bi < ii))
      return acc + beats.astype(jnp.int32)
    rank = pl.loop(0, _K, init_carry=_splat(jnp.int32(0)), unroll=4)(rk_body)
    mask = (rank < _K) & ((_splat(r * 16) + lane) < _splat(jnp.int32(_K)))
    plsc.store_scatter(oval, [rank], ki, mask=mask)
    plsc.store_scatter(oidx, [rank], ii, mask=mask)

  bdesc.wait()
  sw = plsc.load_gather(tsv, [_splat(2 * b + 1)]).astype(jnp.float32)
  sh = plsc.load_gather(tsv, [_splat(2 * b)]).astype(jnp.float32)
  recip_c = jnp.float32(1.0 / _C)
  for r in range(7):
    iv = oidx[pl.ds(r * 16, 16)]
    xf = iv.astype(jnp.float32)
    q = (xf * recip_c + jnp.float32(0.005)).astype(jnp.int32)
    olab[pl.ds(r * 16, 16)] = iv - q * _C
    bq = q * 4
    cx = plsc.load_gather(boxv, [bq])
    cy = plsc.load_gather(boxv, [bq + 1])
    w = plsc.load_gather(boxv, [bq + 2])
    h = plsc.load_gather(boxv, [bq + 3])
    x0 = (cx - 0.5 * w) * sw
    y0 = (cy - 0.5 * h) * sh
    x1 = (cx + 0.5 * w) * sw
    y1 = (cy + 0.5 * h) * sh
    obox[pl.ds(0 * _KPAD + r * 16, 16)] = x0
    obox[pl.ds(1 * _KPAD + r * 16, 16)] = y0
    obox[pl.ds(2 * _KPAD + r * 16, 16)] = x1
    obox[pl.ds(3 * _KPAD + r * 16, 16)] = y1

  pltpu.sync_copy(oval, scores_hbm.at[pl.ds(b * _KPAD, _KPAD)])
  pltpu.sync_copy(olab, labels_hbm.at[pl.ds(b * _KPAD, _KPAD)])
  pltpu.sync_copy(obox, oboxes_hbm.at[pl.ds(b * _KPAD * 4, _KPAD * 4)])


def kernel(pred_logits, pred_boxes, target_sizes, batchsize, num_episode,
           num_queries, num_classes):
  del batchsize, num_episode, num_queries, num_classes
  prob = jax.nn.sigmoid(pred_logits).reshape(-1)
  keys = lax.bitcast_convert_type(prob, jnp.int32)
  boxes_flat = pred_boxes.reshape(-1)
  tsz = target_sizes.reshape(-1).astype(jnp.int32)
  scores_f, labels_f, boxes_f = _postprocess_sc(keys, boxes_flat, tsz)
  scores = lax.bitcast_convert_type(scores_f, jnp.float32).reshape(
      _B, _KPAD)[:, :_K]
  labels = labels_f.reshape(_B, _KPAD)[:, :_K]
  boxes = boxes_f.reshape(_B, 4, _KPAD).transpose(0, 2, 1)[:, :_K, :]
  return scores, labels, boxes
